# baseline (speedup 1.0000x reference)
import jax
import jax.numpy as jnp
from jax import lax
from jax.experimental import pallas as pl
from jax.experimental.pallas import tpu as pltpu
from jax.experimental.pallas import tpu_sc as plsc

NC = 2
NS = 16
L = 16
NW = NC * NS
CH = 96

F32 = jnp.float32


def _zero_row(ref, n):
  zeros = jnp.zeros((L,), F32)

  def body(i, c):
    ref[0, pl.ds(i * L, L)] = zeros
    return c

  lax.fori_loop(0, n // L, body, 0)


def _zero_2d(ref, rows, cols):
  zeros = jnp.zeros((L,), F32)

  def body(i, c):
    for l in range(cols // L):
      ref[i, pl.ds(l * L, L)] = zeros
    return c

  lax.fori_loop(0, rows, body, 0)


def _make_sc_deg(N, E):
  epw = E // NW

  def sc_deg_body(dst_hbm, deg_out, dstf, degv):
    cid = lax.axis_index("c")
    sid = lax.axis_index("s")
    wid = cid * NS + sid
    pltpu.sync_copy(dst_hbm.at[pl.ds(wid * epw, epw)], dstf)
    _zero_row(degv, N)
    ones = jnp.ones((L,), F32)
    z16 = jnp.zeros((L,), jnp.int32)

    def scat(i, c):
      idx = dstf[pl.ds(i * L, L)]
      plsc.addupdate_scatter(degv, [z16, idx], ones)
      return c

    lax.fori_loop(0, epw // L, scat, 0)
    pltpu.sync_copy(degv, deg_out.at[wid])

  mesh = plsc.VectorSubcoreMesh(core_axis_name="c", subcore_axis_name="s")
  return pl.kernel(
      sc_deg_body,
      out_type=jax.ShapeDtypeStruct((NW, 1, N), F32),
      mesh=mesh,
      compiler_params=pltpu.CompilerParams(
          needs_layout_passes=False, use_tc_tiling_on_sc=False),
      scratch_types=[
          pltpu.VMEM((epw,), jnp.int32),
          pltpu.VMEM((1, N), F32),
      ],
  )


def _make_sc_prop(N, D, E, with_sigma):
  epw = E // NW
  epp = -(-epw // CH) * CH
  nch = epp // CH
  hd = D // 2
  npad = -(-N // (128 * NS)) * 128 * NS
  rpt = npad // NS
  zr = rpt // 5

  def sc_prop_body(xta_hbm, xtb_hbm, src_hbm, dst2_hbm, dis_hbm, acc_out,
           sig_out, srcf, dst2v, disv, sigv, rb0, rb1, rb2, rb3, zbuf, accsh,
           g0, g1, g2, g3, s0, s1, s2, s3):
    rbufs = (rb0, rb1, rb2, rb3)
    gsems = (g0, g1, g2, g3)
    ssems = (s0, s1, s2, s3)
    cid = lax.axis_index("c")
    sid = lax.axis_index("s")
    wid = cid * NS + sid
    pltpu.sync_copy(src_hbm.at[wid], srcf)
    pltpu.sync_copy(dst2_hbm.at[wid], dst2v)
    if with_sigma:
      pltpu.sync_copy(dis_hbm, disv.at[pl.ds(0, N)])
      disv[pl.ds(N, L)] = jnp.zeros((L,), F32)
      disv[pl.ds(N + L, L)] = jnp.zeros((L,), F32)

    _zero_2d(zbuf, zr, hd)
    for h, xt_hbm in enumerate((xta_hbm, xtb_hbm)):
      for t in range(5):
        pltpu.sync_copy(zbuf, accsh.at[pl.ds(sid * rpt + t * zr, zr)])
      plsc.subcore_barrier()

      def gather(j, k):
        pltpu.async_copy(xt_hbm.at[srcf.at[pl.ds(j * CH, CH)]], rbufs[k],
                         gsems[k])

      def drain_g(k):
        pltpu.make_async_copy(xt_hbm.at[pl.ds(0, CH)], rbufs[k],
                              gsems[k]).wait()

      def scat(j, k):
        pltpu.async_copy(rbufs[k], accsh.at[dst2v.at[j]], ssems[k], add=True)

      def drain_s(k):
        pltpu.make_async_copy(rbufs[k], accsh.at[pl.ds(0, CH)],
                              ssems[k]).wait()

      gather(0, 0)
      gather(1, 1)
      gather(2, 2)
      drain_g(0)
      scat(0, 0)
      gather(3, 3)
      drain_g(1)
      scat(1, 1)

      def quad(q, c):
        j = 2 + q * 4
        for b in range(4):
          jj = j + b
          kg = b
          drain_s(kg)
          gather(jj + 2, kg)
          k = (2 + b) % 4
          drain_g(k)
          scat(jj, k)
        return c

      lax.fori_loop(0, (nch - 5) // 4, quad, 0)
      for jj in (nch - 3, nch - 2, nch - 1):
        kg = (jj + 2) % 4
        drain_s(kg)
        if jj + 2 < nch:
          gather(jj + 2, kg)
        drain_g(jj % 4)
        scat(jj, jj % 4)
      drain_s((nch - 2) % 4)
      drain_s((nch - 1) % 4)

      if with_sigma and h == 0:
        _zero_row(sigv, N)
        z16 = jnp.zeros((L,), jnp.int32)

        def sg(i, c):
          r = i // (CH // L)
          co = (i % (CH // L)) * L
          vals = plsc.load_gather(disv, [dst2v[r, pl.ds(co, L)]])
          plsc.addupdate_scatter(sigv, [z16, srcf[pl.ds(i * L, L)]], vals)
          return c

        lax.fori_loop(0, epp // L, sg, 0)
        pltpu.sync_copy(sigv, sig_out.at[wid])

      plsc.subcore_barrier()

      for t in range(5):
        r0 = sid * rpt + t * zr
        pltpu.sync_copy(accsh.at[pl.ds(r0, zr)], acc_out.at[cid, h, pl.ds(r0, zr)])

  mesh = plsc.VectorSubcoreMesh(core_axis_name="c", subcore_axis_name="s")
  out_type = (jax.ShapeDtypeStruct((NC, 2, npad, hd), F32),
              jax.ShapeDtypeStruct((NW, 1, N), F32))
  return pl.kernel(
      sc_prop_body,
      out_type=out_type,
      mesh=mesh,
      compiler_params=pltpu.CompilerParams(
          needs_layout_passes=False, use_tc_tiling_on_sc=False),
      scratch_types=[
          pltpu.VMEM((epp,), jnp.int32),
          pltpu.VMEM((nch, CH), jnp.int32),
          pltpu.VMEM((N + 2 * L,), F32),
          pltpu.VMEM((1, N), F32),
          pltpu.VMEM((CH, hd), F32),
          pltpu.VMEM((CH, hd), F32),
          pltpu.VMEM((CH, hd), F32),
          pltpu.VMEM((CH, hd), F32),
          pltpu.VMEM((zr, hd), F32),
          pltpu.VMEM_SHARED((npad, hd), F32),
          pltpu.SemaphoreType.DMA,
          pltpu.SemaphoreType.DMA,
          pltpu.SemaphoreType.DMA,
          pltpu.SemaphoreType.DMA,
          pltpu.SemaphoreType.DMA,
          pltpu.SemaphoreType.DMA,
          pltpu.SemaphoreType.DMA,
          pltpu.SemaphoreType.DMA,
      ],
  )


def _acc_cat(acc_ref, n):
  a = acc_ref[0, 0, :n] + acc_ref[1, 0, :n]
  b = acc_ref[0, 1, :n] + acc_ref[1, 1, :n]
  return jnp.concatenate([a, b], axis=1)


def _tc1_body(degp_ref, x_ref, w_ref, xta_ref, xtb_ref, xl_ref, dis_ref):
  deg = jnp.sum(degp_ref[...], axis=0).reshape(-1, 1) + 1.0
  dis = jax.lax.rsqrt(deg)
  xl = jnp.dot(x_ref[...], w_ref[...], preferred_element_type=F32)
  xl_ref[...] = xl
  xt = xl * dis
  hd = xta_ref.shape[1]
  xta_ref[...] = xt[:, :hd]
  xtb_ref[...] = xt[:, hd:]
  dis_ref[...] = dis


def _tc2_body(acc_ref, xl_ref, dis_ref, w_ref, b_ref, xta_o, xtb_o, xl_o):
  dis = dis_ref[...]
  n = xl_ref.shape[0]
  hd = xta_o.shape[1]
  acc = _acc_cat(acc_ref, n)
  h = jnp.maximum(dis * acc + dis * dis * xl_ref[...] + b_ref[...], 0.0)
  xl2 = jnp.dot(h, w_ref[...], preferred_element_type=F32)
  xl_o[...] = xl2
  xt = xl2 * dis
  xta_o[...] = xt[:, :hd]
  xtb_o[...] = xt[:, hd:]


def _tc3_body(acc_ref, xl_ref, dis_ref, sigp_ref, w_ref, b2_ref, b3_ref,
              out_ref):
  dis = dis_ref[...]
  n = xl_ref.shape[0]
  acc = _acc_cat(acc_ref, n)
  h2 = jnp.maximum(dis * acc + dis * dis * xl_ref[...] + b2_ref[...], 0.0)
  c = dis * jnp.sum(sigp_ref[...], axis=0).reshape(-1, 1) + dis * dis
  r = jnp.sum(c * h2, axis=0, keepdims=True)
  out_ref[...] = jnp.dot(r, w_ref[...], preferred_element_type=F32) * (
      1.0 / n) + b3_ref[...]


@jax.jit
def kernel(x, edge_index, W1, b1, W2, b2, W3, b3):
  N, _ = x.shape
  D_H = W1.shape[1]
  D_OUT = W3.shape[1]
  E = edge_index.shape[1]
  HD = D_H // 2
  src = edge_index[0]
  dst = edge_index[1]
  epw = E // NW
  epp = -(-epw // CH) * CH
  srcp = jnp.pad(src.reshape(NW, epw), ((0, 0), (0, epp - epw)))
  pad_rows = jnp.broadcast_to((N + jnp.arange(NW, dtype=dst.dtype))[:, None],
                              (NW, epp - epw))
  dstp = jnp.concatenate([dst.reshape(NW, epw), pad_rows], axis=1)
  dst2d = dstp.reshape(NW, epp // CH, CH)

  deg_p = _make_sc_deg(N, E)(dst)

  xta1, xtb1, xl1, dis2 = pl.pallas_call(
      _tc1_body,
      out_shape=(jax.ShapeDtypeStruct((N, HD), F32),
                 jax.ShapeDtypeStruct((N, HD), F32),
                 jax.ShapeDtypeStruct((N, D_H), F32),
                 jax.ShapeDtypeStruct((N, 1), F32)),
  )(deg_p.reshape(NW, N), x, W1)

  disf = dis2.reshape(N)
  sc_prop = _make_sc_prop(N, D_H, E, True)
  acc1, sig_p = sc_prop(xta1, xtb1, srcp, dst2d, disf)

  xta2, xtb2, xl2 = pl.pallas_call(
      _tc2_body,
      out_shape=(jax.ShapeDtypeStruct((N, HD), F32),
                 jax.ShapeDtypeStruct((N, HD), F32),
                 jax.ShapeDtypeStruct((N, D_H), F32)),
  )(acc1, xl1, dis2, W2, b1.reshape(1, D_H))

  acc2, _ = sc_prop(xta2, xtb2, srcp, dst2d, disf)

  out = pl.pallas_call(
      _tc3_body,
      out_shape=jax.ShapeDtypeStruct((1, D_OUT), F32),
  )(acc2, xl2, dis2, sig_p.reshape(NW, N), W3, b2.reshape(1, D_H),
    b3.reshape(1, D_OUT))
  return out


# speedup vs baseline: 1.5687x; 1.5687x over previous
import jax
import jax.numpy as jnp
from jax import lax
from jax.experimental import pallas as pl
from jax.experimental.pallas import tpu as pltpu
from jax.experimental.pallas import tpu_sc as plsc

NC = 2
NS = 16
L = 16
NW = NC * NS
CH = 80

F32 = jnp.float32


def _zero_row(ref, n):
  zeros = jnp.zeros((L,), F32)

  def body(i, c):
    ref[0, pl.ds(i * L, L)] = zeros
    return c

  lax.fori_loop(0, n // L, body, 0)


def _zero_2d(ref, rows, cols):
  zeros = jnp.zeros((L,), F32)

  def body(i, c):
    for l in range(cols // L):
      ref[i, pl.ds(l * L, L)] = zeros
    return c

  lax.fori_loop(0, rows, body, 0)


def _make_sc_deg(N, E):
  epw = E // NW

  def sc_deg_body(dst_hbm, deg_out, dstf, degv):
    cid = lax.axis_index("c")
    sid = lax.axis_index("s")
    wid = cid * NS + sid
    pltpu.sync_copy(dst_hbm.at[pl.ds(wid * epw, epw)], dstf)
    _zero_row(degv, N)
    ones = jnp.ones((L,), F32)
    z16 = jnp.zeros((L,), jnp.int32)

    def scat(i, c):
      idx = dstf[pl.ds(i * L, L)]
      plsc.addupdate_scatter(degv, [z16, idx], ones)
      return c

    lax.fori_loop(0, epw // L, scat, 0)
    pltpu.sync_copy(degv, deg_out.at[wid])

  mesh = plsc.VectorSubcoreMesh(core_axis_name="c", subcore_axis_name="s")
  return pl.kernel(
      sc_deg_body,
      out_type=jax.ShapeDtypeStruct((NW, 1, N), F32),
      mesh=mesh,
      compiler_params=pltpu.CompilerParams(
          needs_layout_passes=False, use_tc_tiling_on_sc=False),
      scratch_types=[
          pltpu.VMEM((epw,), jnp.int32),
          pltpu.VMEM((1, N), F32),
      ],
  )


def _make_sc_prop(N, D, E, with_sigma):
  epw = E // NW
  epp = -(-epw // CH) * CH
  nch = epp // CH
  hd = D // 2
  npad = -(-N // (128 * NS)) * 128 * NS
  rpt = npad // NS
  zr = rpt // 5

  def sc_prop_body(xta_hbm, xtb_hbm, src_hbm, dst2_hbm, dis_hbm, acc_out,
           sig_out, srcf, dst2v, disv, sigv, rb0, rb1, rb2, rb3, rb4, rb5,
           zbuf, accsh, g0, g1, g2, g3, g4, g5, s0, s1, s2, s3, s4, s5):
    rbufs = (rb0, rb1, rb2, rb3, rb4, rb5)
    gsems = (g0, g1, g2, g3, g4, g5)
    ssems = (s0, s1, s2, s3, s4, s5)
    cid = lax.axis_index("c")
    sid = lax.axis_index("s")
    wid = cid * NS + sid
    pltpu.sync_copy(src_hbm.at[wid], srcf)
    pltpu.sync_copy(dst2_hbm.at[wid], dst2v)
    if with_sigma:
      pltpu.sync_copy(dis_hbm, disv.at[pl.ds(0, N)])
      disv[pl.ds(N, L)] = jnp.zeros((L,), F32)
      disv[pl.ds(N + L, L)] = jnp.zeros((L,), F32)

    _zero_2d(zbuf, zr, hd)
    for h, xt_hbm in enumerate((xta_hbm, xtb_hbm)):
      for t in range(5):
        pltpu.sync_copy(zbuf, accsh.at[pl.ds(sid * rpt + t * zr, zr)])
      plsc.subcore_barrier()

      def gather(j, k):
        pltpu.async_copy(xt_hbm.at[srcf.at[pl.ds(j * CH, CH)]], rbufs[k],
                         gsems[k])

      def drain_g(k):
        pltpu.make_async_copy(xt_hbm.at[pl.ds(0, CH)], rbufs[k],
                              gsems[k]).wait()

      def scat(j, k):
        pltpu.async_copy(rbufs[k], accsh.at[dst2v.at[j]], ssems[k], add=True)

      def drain_s(k):
        pltpu.make_async_copy(rbufs[k], accsh.at[pl.ds(0, CH)],
                              ssems[k]).wait()

      for p in range(3):
        gather(p, p)
      for p in range(3):
        gather(p + 3, p + 3)
        drain_g(p)
        scat(p, p)

      def hexa(q, c):
        j = 3 + q * 6
        for b in range(6):
          jj = j + b
          kg = b
          drain_s(kg)
          gather(jj + 3, kg)
          k = (3 + b) % 6
          drain_g(k)
          scat(jj, k)
        return c

      lax.fori_loop(0, (nch - 11) // 6, hexa, 0)
      for jj in range(nch - 8, nch):
        kg = (jj + 3) % 6
        drain_s(kg)
        if jj + 3 < nch:
          gather(jj + 3, kg)
        drain_g(jj % 6)
        scat(jj, jj % 6)
      for jj in (nch - 3, nch - 2, nch - 1):
        drain_s(jj % 6)

      if with_sigma and h == 0:
        _zero_row(sigv, N)
        z16 = jnp.zeros((L,), jnp.int32)

        def sg(i, c):
          r = i // (CH // L)
          co = (i % (CH // L)) * L
          vals = plsc.load_gather(disv, [dst2v[r, pl.ds(co, L)]])
          plsc.addupdate_scatter(sigv, [z16, srcf[pl.ds(i * L, L)]], vals)
          return c

        lax.fori_loop(0, epp // L, sg, 0)
        pltpu.sync_copy(sigv, sig_out.at[wid])

      plsc.subcore_barrier()

      for t in range(5):
        r0 = sid * rpt + t * zr
        pltpu.sync_copy(accsh.at[pl.ds(r0, zr)], acc_out.at[cid, h, pl.ds(r0, zr)])

  mesh = plsc.VectorSubcoreMesh(core_axis_name="c", subcore_axis_name="s")
  out_type = (jax.ShapeDtypeStruct((NC, 2, npad, hd), F32),
              jax.ShapeDtypeStruct((NW, 1, N), F32))
  return pl.kernel(
      sc_prop_body,
      out_type=out_type,
      mesh=mesh,
      compiler_params=pltpu.CompilerParams(
          needs_layout_passes=False, use_tc_tiling_on_sc=False),
      scratch_types=[
          pltpu.VMEM((epp,), jnp.int32),
          pltpu.VMEM((nch, CH), jnp.int32),
          pltpu.VMEM((N + 2 * L,), F32),
          pltpu.VMEM((1, N), F32),
          pltpu.VMEM((CH, hd), F32),
          pltpu.VMEM((CH, hd), F32),
          pltpu.VMEM((CH, hd), F32),
          pltpu.VMEM((CH, hd), F32),
          pltpu.VMEM((CH, hd), F32),
          pltpu.VMEM((CH, hd), F32),
          pltpu.VMEM((zr, hd), F32),
          pltpu.VMEM_SHARED((npad, hd), F32),
      ] + [pltpu.SemaphoreType.DMA] * 12,
  )


def _acc_cat(acc_ref, n):
  a = acc_ref[0, 0, :n] + acc_ref[1, 0, :n]
  b = acc_ref[0, 1, :n] + acc_ref[1, 1, :n]
  return jnp.concatenate([a, b], axis=1)


def _tc1_body(degp_ref, x_ref, w_ref, xta_ref, xtb_ref, xl_ref, dis_ref):
  deg = jnp.sum(degp_ref[...], axis=0).reshape(-1, 1) + 1.0
  dis = jax.lax.rsqrt(deg)
  xl = jnp.dot(x_ref[...], w_ref[...], preferred_element_type=F32)
  xl_ref[...] = xl
  xt = xl * dis
  hd = xta_ref.shape[1]
  xta_ref[...] = xt[:, :hd]
  xtb_ref[...] = xt[:, hd:]
  dis_ref[...] = dis


def _tc2_body(acc_ref, xl_ref, dis_ref, w_ref, b_ref, xta_o, xtb_o, xl_o):
  dis = dis_ref[...]
  n = xl_ref.shape[0]
  hd = xta_o.shape[1]
  acc = _acc_cat(acc_ref, n)
  h = jnp.maximum(dis * acc + dis * dis * xl_ref[...] + b_ref[...], 0.0)
  xl2 = jnp.dot(h, w_ref[...], preferred_element_type=F32)
  xl_o[...] = xl2
  xt = xl2 * dis
  xta_o[...] = xt[:, :hd]
  xtb_o[...] = xt[:, hd:]


def _tc3_body(acc_ref, xl_ref, dis_ref, sigp_ref, w_ref, b2_ref, b3_ref,
              out_ref):
  dis = dis_ref[...]
  n = xl_ref.shape[0]
  acc = _acc_cat(acc_ref, n)
  h2 = jnp.maximum(dis * acc + dis * dis * xl_ref[...] + b2_ref[...], 0.0)
  c = dis * jnp.sum(sigp_ref[...], axis=0).reshape(-1, 1) + dis * dis
  r = jnp.sum(c * h2, axis=0, keepdims=True)
  out_ref[...] = jnp.dot(r, w_ref[...], preferred_element_type=F32) * (
      1.0 / n) + b3_ref[...]


@jax.jit
def kernel(x, edge_index, W1, b1, W2, b2, W3, b3):
  N, _ = x.shape
  D_H = W1.shape[1]
  D_OUT = W3.shape[1]
  E = edge_index.shape[1]
  HD = D_H // 2
  src = edge_index[0]
  dst = edge_index[1]
  epw = E // NW
  epp = -(-epw // CH) * CH
  srcp = jnp.pad(src.reshape(NW, epw), ((0, 0), (0, epp - epw)))
  pad_rows = jnp.broadcast_to((N + jnp.arange(NW, dtype=dst.dtype))[:, None],
                              (NW, epp - epw))
  dstp = jnp.concatenate([dst.reshape(NW, epw), pad_rows], axis=1)
  dst2d = dstp.reshape(NW, epp // CH, CH)

  deg_p = _make_sc_deg(N, E)(dst)

  xta1, xtb1, xl1, dis2 = pl.pallas_call(
      _tc1_body,
      out_shape=(jax.ShapeDtypeStruct((N, HD), F32),
                 jax.ShapeDtypeStruct((N, HD), F32),
                 jax.ShapeDtypeStruct((N, D_H), F32),
                 jax.ShapeDtypeStruct((N, 1), F32)),
  )(deg_p.reshape(NW, N), x, W1)

  disf = dis2.reshape(N)
  sc_prop = _make_sc_prop(N, D_H, E, True)
  acc1, sig_p = sc_prop(xta1, xtb1, srcp, dst2d, disf)

  xta2, xtb2, xl2 = pl.pallas_call(
      _tc2_body,
      out_shape=(jax.ShapeDtypeStruct((N, HD), F32),
                 jax.ShapeDtypeStruct((N, HD), F32),
                 jax.ShapeDtypeStruct((N, D_H), F32)),
  )(acc1, xl1, dis2, W2, b1.reshape(1, D_H))

  acc2, _ = sc_prop(xta2, xtb2, srcp, dst2d, disf)

  out = pl.pallas_call(
      _tc3_body,
      out_shape=jax.ShapeDtypeStruct((1, D_OUT), F32),
  )(acc2, xl2, dis2, sig_p.reshape(NW, N), W3, b2.reshape(1, D_H),
    b3.reshape(1, D_OUT))
  return out
